# baseline (speedup 1.0000x reference)
import functools

import jax
import jax.numpy as jnp
from jax import lax
from jax.experimental import pallas as pl
from jax.experimental.pallas import tpu as pltpu
from jax.experimental.pallas import tpu_sc as plsc

B = 16384
DIM = 64
NC = 2
NS = 16
L = 16
NW = NC * NS
BPW = B // NW
CHUNK = 128
NCHUNK = BPW // CHUNK


def _body(s_hbm, p_hbm, o_hbm, nodes_hbm, rel_hbm, out_hbm,
          idx_s, idx_p, idx_o, rows_s, rows_p, rows_o, scores_v, sem):
    wid = lax.axis_index("s") * NC + lax.axis_index("c")
    base = wid * BPW

    pltpu.sync_copy(s_hbm.at[wid], idx_s)
    pltpu.sync_copy(p_hbm.at[wid], idx_p)
    pltpu.sync_copy(o_hbm.at[wid], idx_o)

    copies = []
    for k in range(NCHUNK):
        dst = pl.ds(k * CHUNK, CHUNK)
        copies.append(pltpu.async_copy(nodes_hbm.at[idx_s.at[k]],
                                       rows_s.at[dst], sem))
        copies.append(pltpu.async_copy(rel_hbm.at[idx_p.at[k]],
                                       rows_p.at[dst], sem))
        copies.append(pltpu.async_copy(nodes_hbm.at[idx_o.at[k]],
                                       rows_o.at[dst], sem))
    for c in copies:
        c.wait()

    iota = lax.broadcasted_iota(jnp.int32, (L,), 0)

    def group(g, carry):
        row_idx = g * L + iota
        acc = jnp.zeros((L,), jnp.float32)
        for j in range(DIM):
            col = jnp.full((L,), j, jnp.int32)
            sc = plsc.load_gather(rows_s, [row_idx, col])
            pc = plsc.load_gather(rows_p, [row_idx, col])
            oc = plsc.load_gather(rows_o, [row_idx, col])
            acc = acc + sc * pc * oc
        scores_v[pl.ds(g * L, L)] = acc
        return carry

    lax.fori_loop(0, BPW // L, group, None)

    pltpu.sync_copy(scores_v, out_hbm.at[pl.ds(base, BPW)])


@functools.partial(
    pl.kernel,
    out_type=jax.ShapeDtypeStruct((B,), jnp.float32),
    mesh=plsc.VectorSubcoreMesh(core_axis_name="c", subcore_axis_name="s",
                                num_cores=NC, num_subcores=NS),
    scratch_types=[
        pltpu.VMEM((NCHUNK, CHUNK), jnp.int32),
        pltpu.VMEM((NCHUNK, CHUNK), jnp.int32),
        pltpu.VMEM((NCHUNK, CHUNK), jnp.int32),
        pltpu.VMEM((BPW, DIM), jnp.float32),
        pltpu.VMEM((BPW, DIM), jnp.float32),
        pltpu.VMEM((BPW, DIM), jnp.float32),
        pltpu.VMEM((BPW,), jnp.float32),
        pltpu.SemaphoreType.DMA,
    ],
    compiler_params=pltpu.CompilerParams(needs_layout_passes=False,
                                         use_tc_tiling_on_sc=False),
)
def _distmult_sc(s_hbm, p_hbm, o_hbm, nodes_hbm, rel_hbm, out_hbm, *scratch):
    _body(s_hbm, p_hbm, o_hbm, nodes_hbm, rel_hbm, out_hbm, *scratch)


def kernel(triples, nodes, relations):
    s = triples[:, 0].astype(jnp.int32).reshape(NW, NCHUNK, CHUNK)
    p = triples[:, 1].astype(jnp.int32).reshape(NW, NCHUNK, CHUNK)
    o = triples[:, 2].astype(jnp.int32).reshape(NW, NCHUNK, CHUNK)
    return _distmult_sc(s, p, o, nodes, relations)
